# baseline (speedup 1.0000x reference)
import functools

import jax
import jax.numpy as jnp
from jax import lax
from jax.experimental import pallas as pl
from jax.experimental.pallas import tpu as pltpu
from jax.experimental.pallas import tpu_sc as plsc

N_NODES = 10000
D_IN = 128
D_OUT = 64
E_RAW = 320000

N_WORKERS = 32
WIN = 128
NWIN = 81
EDGES_PER_TILE = WIN * NWIN
E_PAD = EDGES_PER_TILE * N_WORKERS
NPAD = 10240
ROWS_PER_TILE = NPAD // 16


def _tc_prologue(x, W, av8):
    def body(x_ref, w_ref, av_ref, h_ref, a8_ref):
        h = lax.dot_general(x_ref[...], w_ref[...], (((1,), (1,)), ((), ())),
                            preferred_element_type=jnp.float32)
        h_ref[...] = h
        a8_ref[...] = lax.dot_general(av_ref[...], h, (((1,), (1,)), ((), ())),
                                      preferred_element_type=jnp.float32)

    return pl.pallas_call(
        body,
        out_shape=(
            jax.ShapeDtypeStruct((N_NODES, D_OUT), jnp.float32),
            jax.ShapeDtypeStruct((8, N_NODES), jnp.float32),
        ),
    )(x, W, av8)


def _sc_edge_kernel():
    mesh = plsc.VectorSubcoreMesh(core_axis_name="c", subcore_axis_name="s")

    @functools.partial(
        pl.kernel,
        out_type=(
            jax.ShapeDtypeStruct((2, NPAD, D_OUT), jnp.float32),
            jax.ShapeDtypeStruct((2, NPAD), jnp.float32),
        ),
        mesh=mesh,
        compiler_params=pltpu.CompilerParams(
            needs_layout_passes=False, use_tc_tiling_on_sc=False),
        scratch_types=[
            pltpu.VMEM((NWIN, WIN), jnp.int32),
            pltpu.VMEM((NWIN, WIN), jnp.int32),
            pltpu.VMEM((N_NODES,), jnp.float32),
            pltpu.VMEM((N_NODES,), jnp.float32),
            pltpu.VMEM((NWIN, WIN), jnp.float32),
            pltpu.VMEM((2, WIN, D_OUT), jnp.float32),
            pltpu.VMEM_SHARED((NPAD, D_OUT), jnp.float32),
            pltpu.VMEM_SHARED((NPAD,), jnp.float32),
            pltpu.SemaphoreType.DMA,
            pltpu.SemaphoreType.DMA,
            pltpu.SemaphoreType.DMA,
        ],
    )
    def edge_kernel(src_hbm, dst_hbm, as_hbm, ad_hbm, h_hbm, z64_hbm, z1_hbm,
                    num_hbm, den_hbm,
                    srcv, dstv, asv, adv, wv, rowsv, num_sh, den_sh,
                    gsem, ssem, dsem):
        core = lax.axis_index("c")
        sub = lax.axis_index("s")
        wid = sub * 2 + core

        rbase = sub * ROWS_PER_TILE
        pltpu.sync_copy(z64_hbm.at[pl.ds(rbase, ROWS_PER_TILE)],
                        num_sh.at[pl.ds(rbase, ROWS_PER_TILE)])
        pltpu.sync_copy(z1_hbm.at[pl.ds(rbase, ROWS_PER_TILE)],
                        den_sh.at[pl.ds(rbase, ROWS_PER_TILE)])

        pltpu.sync_copy(src_hbm.at[wid], srcv)
        pltpu.sync_copy(dst_hbm.at[wid], dstv)
        pltpu.sync_copy(as_hbm, asv)
        pltpu.sync_copy(ad_hbm, adv)

        pltpu.async_copy(h_hbm.at[srcv.at[0]], rowsv.at[0], gsem)

        def wpass(j, carry):
            for g in range(WIN // 16):
                sl = pl.ds(g * 16, 16)
                a = (plsc.load_gather(asv, [srcv[j, sl]])
                     + plsc.load_gather(adv, [dstv[j, sl]]))
                a = jnp.maximum(a, a * jnp.float32(0.2))
                wv[j, sl] = jnp.exp(a)
            return carry

        lax.fori_loop(0, NWIN, wpass, 0)

        plsc.subcore_barrier()

        def wait_gather(buf):
            pltpu.make_async_copy(h_hbm.at[pl.ds(0, WIN)], rowsv.at[buf],
                                  gsem).wait()

        def wait_num_scatter(buf):
            pltpu.make_async_copy(rowsv.at[buf], num_sh.at[pl.ds(0, WIN)],
                                  ssem).wait()

        def wait_den_scatter():
            pltpu.make_async_copy(wv.at[0], den_sh.at[pl.ds(0, WIN)],
                                  dsem).wait()

        def window(j, carry):
            buf = lax.rem(j, 2)
            wait_gather(buf)
            @pl.when(j >= 1)
            def _():
                wait_num_scatter(1 - buf)
                wait_den_scatter()

            @pl.when(j + 1 < NWIN)
            def _():
                pltpu.async_copy(h_hbm.at[srcv.at[j + 1]], rowsv.at[1 - buf],
                                 gsem)

            pltpu.async_copy(wv.at[j], den_sh.at[dstv.at[j]], dsem, add=True)

            def scale_block(b, c2):
                base = b * 16
                w16 = wv[j, pl.ds(base, 16)]
                for e in range(16):
                    wvec = jnp.full((16,), w16[e])
                    eb = base + e
                    for c in range(D_OUT // 16):
                        csl = pl.ds(c * 16, 16)
                        rowsv[buf, eb, csl] = rowsv[buf, eb, csl] * wvec
                return c2

            lax.fori_loop(0, WIN // 16, scale_block, 0)

            pltpu.async_copy(rowsv.at[buf], num_sh.at[dstv.at[j]], ssem,
                             add=True)
            return carry

        lax.fori_loop(0, NWIN, window, 0)
        wait_num_scatter(lax.rem(NWIN - 1, 2))
        wait_den_scatter()

        plsc.subcore_barrier()

        pltpu.sync_copy(num_sh.at[pl.ds(rbase, ROWS_PER_TILE)],
                        num_hbm.at[core, pl.ds(rbase, ROWS_PER_TILE)])
        pltpu.sync_copy(den_sh.at[pl.ds(rbase, ROWS_PER_TILE)],
                        den_hbm.at[core, pl.ds(rbase, ROWS_PER_TILE)])

    return edge_kernel


def _tc_epilogue(num_p, den_p, bias, gamma, beta):
    def body(num_ref, den_ref, b_ref, g_ref, be_ref, out_ref):
        num = num_ref[0, :N_NODES, :] + num_ref[1, :N_NODES, :]
        den = den_ref[0, :N_NODES] + den_ref[1, :N_NODES]
        pre = num / (den + jnp.float32(1e-16))[:, None] + b_ref[...]
        pre = jnp.maximum(pre, 0.0)
        mean = jnp.mean(pre, axis=0, keepdims=True)
        var = jnp.mean((pre - mean) ** 2, axis=0, keepdims=True)
        out_ref[...] = ((pre - mean) * lax.rsqrt(var + jnp.float32(1e-5))
                        * g_ref[...] + be_ref[...])

    return pl.pallas_call(
        body,
        out_shape=jax.ShapeDtypeStruct((N_NODES, D_OUT), jnp.float32),
    )(num_p, den_p, bias, gamma, beta)


def kernel(x, edge_index, W, att_src, att_dst, bias, gamma, beta):
    av8 = jnp.concatenate(
        [att_src[None, :], att_dst[None, :],
         jnp.zeros((6, D_OUT), jnp.float32)], axis=0)
    h, a8 = _tc_prologue(x, W, av8)
    a_s = a8[0]
    a_d = a8[1]

    loop = jnp.arange(N_NODES, dtype=jnp.int32)
    n_pad = E_PAD - E_RAW - N_NODES
    pad_src = (jnp.arange(n_pad, dtype=jnp.int32) * 41) % N_NODES
    pad_dst = N_NODES + jnp.arange(n_pad, dtype=jnp.int32) % (NPAD - N_NODES)
    src = jnp.concatenate([edge_index[0], loop, pad_src])
    dst = jnp.concatenate([edge_index[1], loop, pad_dst])
    src3 = src.reshape(N_WORKERS, NWIN, WIN)
    dst3 = dst.reshape(N_WORKERS, NWIN, WIN)

    z64 = jnp.zeros((NPAD, D_OUT), jnp.float32)
    z1 = jnp.zeros((NPAD,), jnp.float32)

    num_p, den_p = _sc_edge_kernel()(src3, dst3, a_s, a_d, h, z64, z1)

    return _tc_epilogue(num_p, den_p, bias, gamma, beta)


# speedup vs baseline: 1.4953x; 1.4953x over previous
import functools

import jax
import jax.numpy as jnp
from jax import lax
from jax.experimental import pallas as pl
from jax.experimental.pallas import tpu as pltpu
from jax.experimental.pallas import tpu_sc as plsc

N_NODES = 10000
D_IN = 128
D_OUT = 64
E_RAW = 320000

N_WORKERS = 32
WIN = 80
NWIN = 125
EDGES_PER_TILE = WIN * NWIN
NPAD = 10240
ROWS_PER_TILE = NPAD // 16


def _tc_prologue(x, W, av8):
    def body(x_ref, w_ref, av_ref, h_ref, a8_ref):
        h = lax.dot_general(x_ref[...], w_ref[...], (((1,), (1,)), ((), ())),
                            preferred_element_type=jnp.float32)
        h_ref[...] = h
        a8_ref[...] = lax.dot_general(av_ref[...], h, (((1,), (1,)), ((), ())),
                                      preferred_element_type=jnp.float32)

    return pl.pallas_call(
        body,
        out_shape=(
            jax.ShapeDtypeStruct((N_NODES, D_OUT), jnp.float32),
            jax.ShapeDtypeStruct((8, N_NODES), jnp.float32),
        ),
    )(x, W, av8)


def _sc_edge_kernel():
    mesh = plsc.VectorSubcoreMesh(core_axis_name="c", subcore_axis_name="s")

    @functools.partial(
        pl.kernel,
        out_type=(
            jax.ShapeDtypeStruct((2, NPAD, D_OUT), jnp.float32),
            jax.ShapeDtypeStruct((2, NPAD), jnp.float32),
        ),
        mesh=mesh,
        compiler_params=pltpu.CompilerParams(
            needs_layout_passes=False, use_tc_tiling_on_sc=False),
        scratch_types=[
            pltpu.VMEM((NWIN, WIN), jnp.int32),
            pltpu.VMEM((NWIN, WIN), jnp.int32),
            pltpu.VMEM((N_NODES,), jnp.float32),
            pltpu.VMEM((N_NODES,), jnp.float32),
            pltpu.VMEM((NWIN, WIN), jnp.float32),
            pltpu.VMEM((2, WIN, D_OUT), jnp.float32),
            pltpu.VMEM_SHARED((NPAD, D_OUT), jnp.float32),
            pltpu.VMEM_SHARED((NPAD,), jnp.float32),
            pltpu.SemaphoreType.DMA,
            pltpu.SemaphoreType.DMA,
            pltpu.SemaphoreType.DMA,
        ],
    )
    def edge_kernel(src_hbm, dst_hbm, a8_hbm, h_hbm, z64_hbm, z1_hbm,
                    num_hbm, den_hbm,
                    srcv, dstv, asv, adv, wv, rowsv, num_sh, den_sh,
                    gsem, ssem, dsem):
        core = lax.axis_index("c")
        sub = lax.axis_index("s")
        wid = sub * 2 + core

        rbase = sub * ROWS_PER_TILE
        pltpu.sync_copy(z64_hbm, num_sh.at[pl.ds(rbase, ROWS_PER_TILE)])
        pltpu.sync_copy(z1_hbm, den_sh.at[pl.ds(rbase, ROWS_PER_TILE)])

        pltpu.sync_copy(src_hbm.at[wid], srcv)
        pltpu.sync_copy(dst_hbm.at[wid], dstv)
        pltpu.sync_copy(a8_hbm.at[0], asv)
        pltpu.sync_copy(a8_hbm.at[1], adv)

        pltpu.async_copy(h_hbm.at[srcv.at[0]], rowsv.at[0], gsem)

        def wpass(j, carry):
            for g in range(WIN // 16):
                sl = pl.ds(g * 16, 16)
                a = (plsc.load_gather(asv, [srcv[j, sl]])
                     + plsc.load_gather(adv, [dstv[j, sl]]))
                a = jnp.maximum(a, a * jnp.float32(0.2))
                wv[j, sl] = jnp.exp(a)
            return carry

        lax.fori_loop(0, NWIN, wpass, 0)

        plsc.subcore_barrier()

        def wait_gather(buf):
            pltpu.make_async_copy(h_hbm.at[pl.ds(0, WIN)], rowsv.at[buf],
                                  gsem).wait()

        def wait_num_scatter(buf):
            pltpu.make_async_copy(rowsv.at[buf], num_sh.at[pl.ds(0, WIN)],
                                  ssem).wait()

        def wait_den_scatter():
            pltpu.make_async_copy(wv.at[0], den_sh.at[pl.ds(0, WIN)],
                                  dsem).wait()

        def window(j, carry):
            buf = lax.rem(j, 2)
            wait_gather(buf)
            @pl.when(j >= 1)
            def _():
                wait_num_scatter(1 - buf)
                wait_den_scatter()

            @pl.when(j + 1 < NWIN)
            def _():
                pltpu.async_copy(h_hbm.at[srcv.at[j + 1]], rowsv.at[1 - buf],
                                 gsem)

            pltpu.async_copy(wv.at[j], den_sh.at[dstv.at[j]], dsem, add=True)

            splat_j = jnp.full((16,), j, jnp.int32)

            def scale(e, c2):
                wvec = plsc.load_gather(
                    wv, [splat_j, jnp.full((16,), e, jnp.int32)])
                for c in range(D_OUT // 16):
                    csl = pl.ds(c * 16, 16)
                    rowsv[buf, e, csl] = rowsv[buf, e, csl] * wvec
                return c2

            lax.fori_loop(0, WIN, scale, 0, unroll=2)

            pltpu.async_copy(rowsv.at[buf], num_sh.at[dstv.at[j]], ssem,
                             add=True)
            return carry

        lax.fori_loop(0, NWIN, window, 0)
        wait_num_scatter(lax.rem(NWIN - 1, 2))
        wait_den_scatter()

        plsc.subcore_barrier()

        pltpu.sync_copy(num_sh.at[pl.ds(rbase, ROWS_PER_TILE)],
                        num_hbm.at[core, pl.ds(rbase, ROWS_PER_TILE)])
        pltpu.sync_copy(den_sh.at[pl.ds(rbase, ROWS_PER_TILE)],
                        den_hbm.at[core, pl.ds(rbase, ROWS_PER_TILE)])

    return edge_kernel


def _tc_epilogue(num_p, den_p, h, a8, bias, gamma, beta):
    def body(num_ref, den_ref, h_ref, a8_ref, b_ref, g_ref, be_ref, out_ref):
        a = a8_ref[0, :] + a8_ref[1, :]
        a = jnp.maximum(a, a * jnp.float32(0.2))
        wself = jnp.exp(a)
        num = (num_ref[0, :N_NODES, :] + num_ref[1, :N_NODES, :]
               + wself[:, None] * h_ref[...])
        den = den_ref[0, :N_NODES] + den_ref[1, :N_NODES] + wself
        pre = num / (den + jnp.float32(1e-16))[:, None] + b_ref[...]
        pre = jnp.maximum(pre, 0.0)
        mean = jnp.mean(pre, axis=0, keepdims=True)
        var = jnp.mean((pre - mean) ** 2, axis=0, keepdims=True)
        out_ref[...] = ((pre - mean) * lax.rsqrt(var + jnp.float32(1e-5))
                        * g_ref[...] + be_ref[...])

    return pl.pallas_call(
        body,
        out_shape=jax.ShapeDtypeStruct((N_NODES, D_OUT), jnp.float32),
    )(num_p, den_p, h, a8, bias, gamma, beta)


def kernel(x, edge_index, W, att_src, att_dst, bias, gamma, beta):
    av8 = jnp.concatenate(
        [att_src[None, :], att_dst[None, :],
         jnp.zeros((6, D_OUT), jnp.float32)], axis=0)
    h, a8 = _tc_prologue(x, W, av8)

    src3 = edge_index[0].reshape(N_WORKERS, NWIN, WIN)
    dst3 = edge_index[1].reshape(N_WORKERS, NWIN, WIN)

    z64 = jnp.zeros((ROWS_PER_TILE, D_OUT), jnp.float32)
    z1 = jnp.zeros((ROWS_PER_TILE,), jnp.float32)

    num_p, den_p = _sc_edge_kernel()(src3, dst3, a8, h, z64, z1)

    return _tc_epilogue(num_p, den_p, h, a8, bias, gamma, beta)


# speedup vs baseline: 1.5054x; 1.0068x over previous
import functools

import jax
import jax.numpy as jnp
from jax import lax
from jax.experimental import pallas as pl
from jax.experimental.pallas import tpu as pltpu
from jax.experimental.pallas import tpu_sc as plsc

N_NODES = 10000
D_IN = 128
D_OUT = 64
E_RAW = 320000

N_WORKERS = 32
WIN = 80
NWIN = 125
EDGES_PER_TILE = WIN * NWIN
NPAD = 10240
ROWS_PER_TILE = NPAD // 16


def _tc_prologue(x, W, av8):
    def body(x_ref, w_ref, av_ref, h_ref, a8_ref):
        h = lax.dot_general(x_ref[...], w_ref[...], (((1,), (1,)), ((), ())),
                            preferred_element_type=jnp.float32)
        h_ref[...] = h
        a8_ref[...] = lax.dot_general(av_ref[...], h, (((1,), (1,)), ((), ())),
                                      preferred_element_type=jnp.float32)

    return pl.pallas_call(
        body,
        out_shape=(
            jax.ShapeDtypeStruct((N_NODES, D_OUT), jnp.float32),
            jax.ShapeDtypeStruct((8, N_NODES), jnp.float32),
        ),
    )(x, W, av8)


def _sc_edge_kernel():
    mesh = plsc.VectorSubcoreMesh(core_axis_name="c", subcore_axis_name="s")

    @functools.partial(
        pl.kernel,
        out_type=(
            jax.ShapeDtypeStruct((2, NPAD, D_OUT), jnp.float32),
            jax.ShapeDtypeStruct((2, NPAD), jnp.float32),
        ),
        mesh=mesh,
        compiler_params=pltpu.CompilerParams(
            needs_layout_passes=False, use_tc_tiling_on_sc=False),
        scratch_types=[
            pltpu.VMEM((NWIN, WIN), jnp.int32),
            pltpu.VMEM((NWIN, WIN), jnp.int32),
            pltpu.VMEM((N_NODES,), jnp.float32),
            pltpu.VMEM((N_NODES,), jnp.float32),
            pltpu.VMEM((NWIN, WIN), jnp.float32),
            pltpu.VMEM((2, WIN, D_OUT), jnp.float32),
            pltpu.VMEM_SHARED((NPAD, D_OUT), jnp.float32),
            pltpu.VMEM_SHARED((NPAD,), jnp.float32),
            pltpu.SemaphoreType.DMA,
            pltpu.SemaphoreType.DMA,
            pltpu.SemaphoreType.DMA,
        ],
    )
    def edge_kernel(src_hbm, dst_hbm, a8_hbm, h_hbm, z64_hbm, z1_hbm,
                    num_hbm, den_hbm,
                    srcv, dstv, asv, adv, wv, rowsv, num_sh, den_sh,
                    gsem, ssem, dsem):
        core = lax.axis_index("c")
        sub = lax.axis_index("s")
        wid = sub * 2 + core

        rbase = sub * ROWS_PER_TILE
        pltpu.sync_copy(z64_hbm, num_sh.at[pl.ds(rbase, ROWS_PER_TILE)])
        pltpu.sync_copy(z1_hbm, den_sh.at[pl.ds(rbase, ROWS_PER_TILE)])

        pltpu.async_copy(src_hbm.at[wid], srcv, gsem)
        pltpu.async_copy(dst_hbm.at[wid], dstv, gsem)
        pltpu.async_copy(a8_hbm.at[0], asv, gsem)
        pltpu.async_copy(a8_hbm.at[1], adv, gsem)
        pltpu.make_async_copy(a8_hbm.at[1], adv, gsem).wait()
        pltpu.make_async_copy(a8_hbm.at[1], adv, gsem).wait()
        pltpu.make_async_copy(src_hbm.at[wid], srcv, gsem).wait()
        pltpu.make_async_copy(src_hbm.at[wid], srcv, gsem).wait()

        pltpu.async_copy(h_hbm.at[srcv.at[0]], rowsv.at[0], gsem)

        def wrow(j):
            for g in range(WIN // 16):
                sl = pl.ds(g * 16, 16)
                a = (plsc.load_gather(asv, [srcv[j, sl]])
                     + plsc.load_gather(adv, [dstv[j, sl]]))
                a = jnp.maximum(a, a * jnp.float32(0.2))
                wv[j, sl] = jnp.exp(a)

        wrow(0)

        plsc.subcore_barrier()

        def wait_gather(buf):
            pltpu.make_async_copy(h_hbm.at[pl.ds(0, WIN)], rowsv.at[buf],
                                  gsem).wait()

        def wait_num_scatter(buf):
            pltpu.make_async_copy(rowsv.at[buf], num_sh.at[pl.ds(0, WIN)],
                                  ssem).wait()

        def wait_den_scatter():
            pltpu.make_async_copy(wv.at[0], den_sh.at[pl.ds(0, WIN)],
                                  dsem).wait()

        def window(j, carry):
            buf = lax.rem(j, 2)
            wait_gather(buf)
            @pl.when(j >= 1)
            def _():
                wait_num_scatter(1 - buf)
                wait_den_scatter()

            @pl.when(j + 1 < NWIN)
            def _():
                pltpu.async_copy(h_hbm.at[srcv.at[j + 1]], rowsv.at[1 - buf],
                                 gsem)
                wrow(j + 1)

            pltpu.async_copy(wv.at[j], den_sh.at[dstv.at[j]], dsem, add=True)

            splat_j = jnp.full((16,), j, jnp.int32)

            def scale(e, c2):
                wvec = plsc.load_gather(
                    wv, [splat_j, jnp.full((16,), e, jnp.int32)])
                for c in range(D_OUT // 16):
                    csl = pl.ds(c * 16, 16)
                    rowsv[buf, e, csl] = rowsv[buf, e, csl] * wvec
                return c2

            lax.fori_loop(0, WIN, scale, 0, unroll=2)

            pltpu.async_copy(rowsv.at[buf], num_sh.at[dstv.at[j]], ssem,
                             add=True)
            return carry

        lax.fori_loop(0, NWIN, window, 0)
        wait_num_scatter(lax.rem(NWIN - 1, 2))
        wait_den_scatter()

        plsc.subcore_barrier()

        pltpu.sync_copy(num_sh.at[pl.ds(rbase, ROWS_PER_TILE)],
                        num_hbm.at[core, pl.ds(rbase, ROWS_PER_TILE)])
        pltpu.sync_copy(den_sh.at[pl.ds(rbase, ROWS_PER_TILE)],
                        den_hbm.at[core, pl.ds(rbase, ROWS_PER_TILE)])

    return edge_kernel


def _tc_epilogue(num_p, den_p, h, a8, bias, gamma, beta):
    def body(num_ref, den_ref, h_ref, a8_ref, b_ref, g_ref, be_ref, out_ref):
        a = a8_ref[0, :] + a8_ref[1, :]
        a = jnp.maximum(a, a * jnp.float32(0.2))
        wself = jnp.exp(a)
        num = (num_ref[0, :N_NODES, :] + num_ref[1, :N_NODES, :]
               + wself[:, None] * h_ref[...])
        den = den_ref[0, :N_NODES] + den_ref[1, :N_NODES] + wself
        pre = num / (den + jnp.float32(1e-16))[:, None] + b_ref[...]
        pre = jnp.maximum(pre, 0.0)
        mean = jnp.mean(pre, axis=0, keepdims=True)
        var = jnp.mean((pre - mean) ** 2, axis=0, keepdims=True)
        out_ref[...] = ((pre - mean) * lax.rsqrt(var + jnp.float32(1e-5))
                        * g_ref[...] + be_ref[...])

    return pl.pallas_call(
        body,
        out_shape=jax.ShapeDtypeStruct((N_NODES, D_OUT), jnp.float32),
    )(num_p, den_p, h, a8, bias, gamma, beta)


def kernel(x, edge_index, W, att_src, att_dst, bias, gamma, beta):
    av8 = jnp.concatenate(
        [att_src[None, :], att_dst[None, :],
         jnp.zeros((6, D_OUT), jnp.float32)], axis=0)
    h, a8 = _tc_prologue(x, W, av8)

    src3 = edge_index[0].reshape(N_WORKERS, NWIN, WIN)
    dst3 = edge_index[1].reshape(N_WORKERS, NWIN, WIN)

    z64 = jnp.zeros((ROWS_PER_TILE, D_OUT), jnp.float32)
    z1 = jnp.zeros((ROWS_PER_TILE,), jnp.float32)

    num_p, den_p = _sc_edge_kernel()(src3, dst3, a8, h, z64, z1)

    return _tc_epilogue(num_p, den_p, h, a8, bias, gamma, beta)


# speedup vs baseline: 1.5214x; 1.0106x over previous
import functools

import jax
import jax.numpy as jnp
from jax import lax
from jax.experimental import pallas as pl
from jax.experimental.pallas import tpu as pltpu
from jax.experimental.pallas import tpu_sc as plsc

N_NODES = 10000
D_IN = 128
D_OUT = 64
E_RAW = 320000

N_WORKERS = 32
WIN = 80
NWIN = 125
EDGES_PER_TILE = WIN * NWIN
NPAD = 10240
ROWS_PER_TILE = NPAD // 16


def _tc_prologue(x, W, av8):
    def body(x_ref, w_ref, av_ref, h_ref, a8_ref):
        h = lax.dot_general(x_ref[...], w_ref[...], (((1,), (1,)), ((), ())),
                            preferred_element_type=jnp.float32)
        h_ref[...] = h
        a8_ref[...] = lax.dot_general(av_ref[...], h, (((1,), (1,)), ((), ())),
                                      preferred_element_type=jnp.float32)

    return pl.pallas_call(
        body,
        out_shape=(
            jax.ShapeDtypeStruct((N_NODES, D_OUT), jnp.float32),
            jax.ShapeDtypeStruct((8, N_NODES), jnp.float32),
        ),
    )(x, W, av8)


def _sc_edge_kernel():
    mesh = plsc.VectorSubcoreMesh(core_axis_name="c", subcore_axis_name="s")

    @functools.partial(
        pl.kernel,
        out_type=(
            jax.ShapeDtypeStruct((2, NPAD, D_OUT), jnp.float32),
            jax.ShapeDtypeStruct((2, NPAD), jnp.float32),
        ),
        mesh=mesh,
        compiler_params=pltpu.CompilerParams(
            needs_layout_passes=False, use_tc_tiling_on_sc=False),
        scratch_types=[
            pltpu.VMEM((NWIN, WIN), jnp.int32),
            pltpu.VMEM((NWIN, WIN), jnp.int32),
            pltpu.VMEM((2, WIN), jnp.float32),
            pltpu.VMEM((2, WIN), jnp.float32),
            pltpu.VMEM((NWIN, WIN), jnp.float32),
            pltpu.VMEM((2, WIN, D_OUT), jnp.float32),
            pltpu.VMEM_SHARED((NPAD, D_OUT), jnp.float32),
            pltpu.VMEM_SHARED((NPAD,), jnp.float32),
            pltpu.VMEM_SHARED((N_NODES, D_OUT), jnp.float32),
            pltpu.VMEM_SHARED((N_NODES,), jnp.float32),
            pltpu.VMEM_SHARED((N_NODES,), jnp.float32),
            pltpu.SemaphoreType.DMA,
            pltpu.SemaphoreType.DMA,
            pltpu.SemaphoreType.DMA,
            pltpu.SemaphoreType.DMA,
        ],
    )
    def edge_kernel(src_hbm, dst_hbm, a8_hbm, h_hbm, z64_hbm, z1_hbm,
                    num_hbm, den_hbm,
                    srcv, dstv, asl, adl, wv, rowsv, num_sh, den_sh, h_sh,
                    as_sh, ad_sh, gsem, ssem, dsem, lsem):
        core = lax.axis_index("c")
        sub = lax.axis_index("s")
        wid = sub * 2 + core

        rbase = sub * ROWS_PER_TILE
        pltpu.sync_copy(z64_hbm, num_sh.at[pl.ds(rbase, ROWS_PER_TILE)])
        pltpu.sync_copy(z1_hbm, den_sh.at[pl.ds(rbase, ROWS_PER_TILE)])

        hrows = N_NODES // 16
        hbase = sub * hrows
        pltpu.async_copy(src_hbm.at[wid], srcv, gsem)
        pltpu.async_copy(dst_hbm.at[wid], dstv, gsem)
        pltpu.async_copy(h_hbm.at[pl.ds(hbase, hrows)],
                         h_sh.at[pl.ds(hbase, hrows)], gsem)

        @pl.when(sub == 0)
        def _():
            pltpu.async_copy(a8_hbm.at[0], as_sh, lsem)

        @pl.when(sub == 1)
        def _():
            pltpu.async_copy(a8_hbm.at[1], ad_sh, lsem)

        pltpu.make_async_copy(src_hbm.at[wid], srcv, gsem).wait()
        pltpu.make_async_copy(src_hbm.at[wid], srcv, gsem).wait()
        pltpu.make_async_copy(h_hbm.at[pl.ds(hbase, hrows)],
                              h_sh.at[pl.ds(hbase, hrows)], gsem).wait()

        @pl.when(sub < 2)
        def _():
            pltpu.make_async_copy(a8_hbm.at[0], as_sh, lsem).wait()

        plsc.subcore_barrier()

        def wissue(j, ring):
            pltpu.async_copy(as_sh.at[srcv.at[j]], asl.at[ring], lsem)
            pltpu.async_copy(ad_sh.at[dstv.at[j]], adl.at[ring], lsem)

        def wrow(j, ring):
            pltpu.make_async_copy(as_sh.at[pl.ds(0, WIN)], asl.at[ring],
                                  lsem).wait()
            pltpu.make_async_copy(as_sh.at[pl.ds(0, WIN)], asl.at[ring],
                                  lsem).wait()
            for g in range(WIN // 16):
                sl = pl.ds(g * 16, 16)
                a = asl[ring, sl] + adl[ring, sl]
                a = jnp.maximum(a, a * jnp.float32(0.2))
                wv[j, sl] = jnp.exp(a)

        pltpu.async_copy(h_sh.at[srcv.at[0]], rowsv.at[0], gsem)
        wissue(0, 0)
        wrow(0, 0)

        def wait_gather(buf):
            pltpu.make_async_copy(h_sh.at[pl.ds(0, WIN)], rowsv.at[buf],
                                  gsem).wait()

        def wait_num_scatter(buf):
            pltpu.make_async_copy(rowsv.at[buf], num_sh.at[pl.ds(0, WIN)],
                                  ssem).wait()

        def wait_den_scatter():
            pltpu.make_async_copy(wv.at[0], den_sh.at[pl.ds(0, WIN)],
                                  dsem).wait()

        def window(j, carry):
            buf = lax.rem(j, 2)
            ring = lax.rem(j + 1, 2)
            wait_gather(buf)
            @pl.when(j >= 1)
            def _():
                wait_num_scatter(1 - buf)
                wait_den_scatter()

            @pl.when(j + 1 < NWIN)
            def _():
                pltpu.async_copy(h_sh.at[srcv.at[j + 1]], rowsv.at[1 - buf],
                                 gsem)
                wissue(j + 1, ring)

            pltpu.async_copy(wv.at[j], den_sh.at[dstv.at[j]], dsem, add=True)

            splat_j = jnp.full((16,), j, jnp.int32)

            def scale(e, c2):
                wvec = plsc.load_gather(
                    wv, [splat_j, jnp.full((16,), e, jnp.int32)])
                for c in range(D_OUT // 16):
                    csl = pl.ds(c * 16, 16)
                    rowsv[buf, e, csl] = rowsv[buf, e, csl] * wvec
                return c2

            lax.fori_loop(0, WIN, scale, 0, unroll=2)

            @pl.when(j + 1 < NWIN)
            def _():
                wrow(j + 1, ring)

            pltpu.async_copy(rowsv.at[buf], num_sh.at[dstv.at[j]], ssem,
                             add=True)
            return carry

        lax.fori_loop(0, NWIN, window, 0)
        wait_num_scatter(lax.rem(NWIN - 1, 2))
        wait_den_scatter()

        plsc.subcore_barrier()

        pltpu.sync_copy(num_sh.at[pl.ds(rbase, ROWS_PER_TILE)],
                        num_hbm.at[core, pl.ds(rbase, ROWS_PER_TILE)])
        pltpu.sync_copy(den_sh.at[pl.ds(rbase, ROWS_PER_TILE)],
                        den_hbm.at[core, pl.ds(rbase, ROWS_PER_TILE)])

    return edge_kernel


def _tc_epilogue(num_p, den_p, h, a8, bias, gamma, beta):
    def body(num_ref, den_ref, h_ref, a8_ref, b_ref, g_ref, be_ref, out_ref):
        a = a8_ref[0, :] + a8_ref[1, :]
        a = jnp.maximum(a, a * jnp.float32(0.2))
        wself = jnp.exp(a)
        num = (num_ref[0, :N_NODES, :] + num_ref[1, :N_NODES, :]
               + wself[:, None] * h_ref[...])
        den = den_ref[0, :N_NODES] + den_ref[1, :N_NODES] + wself
        pre = num / (den + jnp.float32(1e-16))[:, None] + b_ref[...]
        pre = jnp.maximum(pre, 0.0)
        mean = jnp.mean(pre, axis=0, keepdims=True)
        var = jnp.mean((pre - mean) ** 2, axis=0, keepdims=True)
        out_ref[...] = ((pre - mean) * lax.rsqrt(var + jnp.float32(1e-5))
                        * g_ref[...] + be_ref[...])

    return pl.pallas_call(
        body,
        out_shape=jax.ShapeDtypeStruct((N_NODES, D_OUT), jnp.float32),
    )(num_p, den_p, h, a8, bias, gamma, beta)


def kernel(x, edge_index, W, att_src, att_dst, bias, gamma, beta):
    av8 = jnp.concatenate(
        [att_src[None, :], att_dst[None, :],
         jnp.zeros((6, D_OUT), jnp.float32)], axis=0)
    h, a8 = _tc_prologue(x, W, av8)

    src3 = edge_index[0].reshape(N_WORKERS, NWIN, WIN)
    dst3 = edge_index[1].reshape(N_WORKERS, NWIN, WIN)

    z64 = jnp.zeros((ROWS_PER_TILE, D_OUT), jnp.float32)
    z1 = jnp.zeros((ROWS_PER_TILE,), jnp.float32)

    num_p, den_p = _sc_edge_kernel()(src3, dst3, a8, h, z64, z1)

    return _tc_epilogue(num_p, den_p, h, a8, bias, gamma, beta)


# speedup vs baseline: 1.5631x; 1.0274x over previous
import functools

import jax
import jax.numpy as jnp
from jax import lax
from jax.experimental import pallas as pl
from jax.experimental.pallas import tpu as pltpu
from jax.experimental.pallas import tpu_sc as plsc

N_NODES = 10000
D_IN = 128
D_OUT = 64
E_RAW = 320000

N_WORKERS = 32
WIN = 80
NWIN = 125
EDGES_PER_TILE = WIN * NWIN
NPAD = 10240
ROWS_PER_TILE = NPAD // 16


def _tc_prologue(x, W, av8):
    def body(x_ref, w_ref, av_ref, h_ref, a8_ref):
        h = lax.dot_general(x_ref[...], w_ref[...], (((1,), (1,)), ((), ())),
                            preferred_element_type=jnp.float32)
        h_ref[...] = h
        a8_ref[...] = lax.dot_general(av_ref[...], h, (((1,), (1,)), ((), ())),
                                      preferred_element_type=jnp.float32)

    return pl.pallas_call(
        body,
        out_shape=(
            jax.ShapeDtypeStruct((N_NODES, D_OUT), jnp.float32),
            jax.ShapeDtypeStruct((8, N_NODES), jnp.float32),
        ),
    )(x, W, av8)


def _sc_edge_kernel():
    mesh = plsc.VectorSubcoreMesh(core_axis_name="c", subcore_axis_name="s")

    @functools.partial(
        pl.kernel,
        out_type=(
            jax.ShapeDtypeStruct((2, NPAD, D_OUT), jnp.float32),
            jax.ShapeDtypeStruct((2, NPAD), jnp.float32),
        ),
        mesh=mesh,
        compiler_params=pltpu.CompilerParams(
            needs_layout_passes=False, use_tc_tiling_on_sc=False),
        scratch_types=[
            pltpu.VMEM((NWIN, WIN), jnp.int32),
            pltpu.VMEM((NWIN, WIN), jnp.int32),
            pltpu.VMEM((2, WIN), jnp.float32),
            pltpu.VMEM((2, WIN), jnp.float32),
            pltpu.VMEM((2, WIN), jnp.float32),
            pltpu.VMEM((4, WIN, D_OUT), jnp.float32),
            pltpu.VMEM_SHARED((NPAD, D_OUT), jnp.float32),
            pltpu.VMEM_SHARED((NPAD,), jnp.float32),
            pltpu.VMEM_SHARED((N_NODES, D_OUT), jnp.float32),
            pltpu.VMEM_SHARED((N_NODES,), jnp.float32),
            pltpu.VMEM_SHARED((N_NODES,), jnp.float32),
            pltpu.SemaphoreType.DMA,
            pltpu.SemaphoreType.DMA,
            pltpu.SemaphoreType.DMA,
            pltpu.SemaphoreType.DMA,
        ],
    )
    def edge_kernel(src_hbm, dst_hbm, a8_hbm, h_hbm, z64_hbm, z1_hbm,
                    num_hbm, den_hbm,
                    srcv, dstv, asl, adl, wv, rowsv, num_sh, den_sh, h_sh,
                    as_sh, ad_sh, gsem, ssem, dsem, lsem):
        core = lax.axis_index("c")
        sub = lax.axis_index("s")
        wid = sub * 2 + core

        rbase = sub * ROWS_PER_TILE
        pltpu.sync_copy(z64_hbm, num_sh.at[pl.ds(rbase, ROWS_PER_TILE)])
        pltpu.sync_copy(z1_hbm, den_sh.at[pl.ds(rbase, ROWS_PER_TILE)])

        hrows = N_NODES // 16
        hbase = sub * hrows
        pltpu.async_copy(src_hbm.at[wid], srcv, gsem)
        pltpu.async_copy(dst_hbm.at[wid], dstv, gsem)
        pltpu.async_copy(h_hbm.at[pl.ds(hbase, hrows)],
                         h_sh.at[pl.ds(hbase, hrows)], gsem)

        @pl.when(sub == 0)
        def _():
            pltpu.async_copy(a8_hbm.at[0], as_sh, lsem)

        @pl.when(sub == 1)
        def _():
            pltpu.async_copy(a8_hbm.at[1], ad_sh, lsem)

        pltpu.make_async_copy(src_hbm.at[wid], srcv, gsem).wait()
        pltpu.make_async_copy(src_hbm.at[wid], srcv, gsem).wait()
        pltpu.make_async_copy(h_hbm.at[pl.ds(hbase, hrows)],
                              h_sh.at[pl.ds(hbase, hrows)], gsem).wait()

        @pl.when(sub < 2)
        def _():
            pltpu.make_async_copy(a8_hbm.at[0], as_sh, lsem).wait()

        plsc.subcore_barrier()

        def wissue(j, ring):
            pltpu.async_copy(as_sh.at[srcv.at[j]], asl.at[ring], lsem)
            pltpu.async_copy(ad_sh.at[dstv.at[j]], adl.at[ring], lsem)

        def wrow(j, ring):
            pltpu.make_async_copy(as_sh.at[pl.ds(0, WIN)], asl.at[ring],
                                  lsem).wait()
            pltpu.make_async_copy(as_sh.at[pl.ds(0, WIN)], asl.at[ring],
                                  lsem).wait()
            for g in range(WIN // 16):
                sl = pl.ds(g * 16, 16)
                a = asl[ring, sl] + adl[ring, sl]
                a = jnp.maximum(a, a * jnp.float32(0.2))
                wv[ring, sl] = jnp.exp(a)

        pltpu.async_copy(h_sh.at[srcv.at[0]], rowsv.at[0], gsem)
        pltpu.async_copy(h_sh.at[srcv.at[1]], rowsv.at[1], gsem)
        pltpu.async_copy(h_sh.at[srcv.at[2]], rowsv.at[2], gsem)
        wissue(0, 0)
        wrow(0, 0)

        def wait_gather(buf):
            pltpu.make_async_copy(h_sh.at[pl.ds(0, WIN)], rowsv.at[buf],
                                  gsem).wait()

        def wait_num_scatter(buf):
            pltpu.make_async_copy(rowsv.at[buf], num_sh.at[pl.ds(0, WIN)],
                                  ssem).wait()

        def wait_den_scatter():
            pltpu.make_async_copy(wv.at[0], den_sh.at[pl.ds(0, WIN)],
                                  dsem).wait()

        def window(j, carry):
            buf = lax.rem(j, 4)
            wring = lax.rem(j, 2)
            ring = lax.rem(j + 1, 2)
            wait_gather(buf)
            @pl.when(j >= 1)
            def _():
                wait_num_scatter(0)
                wait_den_scatter()

            @pl.when(j + 3 < NWIN)
            def _():
                pltpu.async_copy(h_sh.at[srcv.at[j + 3]],
                                 rowsv.at[lax.rem(j + 3, 4)], gsem)

            @pl.when(j + 1 < NWIN)
            def _():
                wissue(j + 1, ring)

            pltpu.async_copy(wv.at[wring], den_sh.at[dstv.at[j]], dsem,
                             add=True)

            splat_w = jnp.full((16,), wring, jnp.int32)

            def scale(e, c2):
                wvec = plsc.load_gather(
                    wv, [splat_w, jnp.full((16,), e, jnp.int32)])
                for c in range(D_OUT // 16):
                    csl = pl.ds(c * 16, 16)
                    rowsv[buf, e, csl] = rowsv[buf, e, csl] * wvec
                return c2

            lax.fori_loop(0, WIN, scale, 0, unroll=2)

            @pl.when(j + 1 < NWIN)
            def _():
                wrow(j + 1, ring)

            pltpu.async_copy(rowsv.at[buf], num_sh.at[dstv.at[j]], ssem,
                             add=True)
            return carry

        lax.fori_loop(0, NWIN, window, 0)
        wait_num_scatter(0)
        wait_den_scatter()

        plsc.subcore_barrier()

        pltpu.sync_copy(num_sh.at[pl.ds(rbase, ROWS_PER_TILE)],
                        num_hbm.at[core, pl.ds(rbase, ROWS_PER_TILE)])
        pltpu.sync_copy(den_sh.at[pl.ds(rbase, ROWS_PER_TILE)],
                        den_hbm.at[core, pl.ds(rbase, ROWS_PER_TILE)])

    return edge_kernel


def _tc_epilogue(num_p, den_p, h, a8, bias, gamma, beta):
    def body(num_ref, den_ref, h_ref, a8_ref, b_ref, g_ref, be_ref, out_ref):
        a = a8_ref[0, :] + a8_ref[1, :]
        a = jnp.maximum(a, a * jnp.float32(0.2))
        wself = jnp.exp(a)
        num = (num_ref[0, :N_NODES, :] + num_ref[1, :N_NODES, :]
               + wself[:, None] * h_ref[...])
        den = den_ref[0, :N_NODES] + den_ref[1, :N_NODES] + wself
        pre = num / (den + jnp.float32(1e-16))[:, None] + b_ref[...]
        pre = jnp.maximum(pre, 0.0)
        mean = jnp.mean(pre, axis=0, keepdims=True)
        var = jnp.mean((pre - mean) ** 2, axis=0, keepdims=True)
        out_ref[...] = ((pre - mean) * lax.rsqrt(var + jnp.float32(1e-5))
                        * g_ref[...] + be_ref[...])

    return pl.pallas_call(
        body,
        out_shape=jax.ShapeDtypeStruct((N_NODES, D_OUT), jnp.float32),
    )(num_p, den_p, h, a8, bias, gamma, beta)


def kernel(x, edge_index, W, att_src, att_dst, bias, gamma, beta):
    av8 = jnp.concatenate(
        [att_src[None, :], att_dst[None, :],
         jnp.zeros((6, D_OUT), jnp.float32)], axis=0)
    h, a8 = _tc_prologue(x, W, av8)

    src3 = edge_index[0].reshape(N_WORKERS, NWIN, WIN)
    dst3 = edge_index[1].reshape(N_WORKERS, NWIN, WIN)

    z64 = jnp.zeros((ROWS_PER_TILE, D_OUT), jnp.float32)
    z1 = jnp.zeros((ROWS_PER_TILE,), jnp.float32)

    num_p, den_p = _sc_edge_kernel()(src3, dst3, a8, h, z64, z1)

    return _tc_epilogue(num_p, den_p, h, a8, bias, gamma, beta)
